# baseline (speedup 1.0000x reference)
import functools

import jax
import jax.numpy as jnp
from jax import lax
from jax.experimental import pallas as pl
from jax.experimental.pallas import tpu as pltpu
from jax.experimental.pallas import tpu_sc as plsc

GROUP_NUM = 1000000
EMBED = 64
LATENT = 128
BATCH = 16384

_NC = 2
_NS = 16
_NW = _NC * _NS
_B_PER_W = BATCH // _NW
_CHUNK = 128
_NCH = _B_PER_W // _CHUNK


def _sc_gather(table, idx3):
    mesh = plsc.VectorSubcoreMesh(core_axis_name="c", subcore_axis_name="s")

    @functools.partial(
        pl.kernel,
        mesh=mesh,
        compiler_params=pltpu.CompilerParams(use_tc_tiling_on_sc=False),
        out_type=jax.ShapeDtypeStruct((BATCH, EMBED), jnp.float32),
        scratch_types=[
            pltpu.VMEM((_NCH, _CHUNK), jnp.int32),
            pltpu.VMEM((_B_PER_W, EMBED), jnp.float32),
            pltpu.SemaphoreType.DMA,
        ],
    )
    def k(table_hbm, idx_hbm, out_hbm, idx_v, rows_v, sem):
        wid = lax.axis_index("s") * _NC + lax.axis_index("c")
        base = wid * _B_PER_W
        pltpu.sync_copy(idx_hbm.at[wid], idx_v)
        copies = []
        for j in range(_NCH):
            copies.append(
                pltpu.async_copy(
                    table_hbm.at[idx_v.at[j]],
                    rows_v.at[pl.ds(j * _CHUNK, _CHUNK)],
                    sem,
                )
            )
        for c in copies:
            c.wait()
        pltpu.sync_copy(rows_v, out_hbm.at[pl.ds(base, _B_PER_W)])

    return k(table, idx3)


def _fuse_body(g_ref, t_ref, v_ref, w1_ref, w2_ref, w3_ref, b_ref, o_ref):
    acc = jnp.dot(g_ref[...], w1_ref[...], preferred_element_type=jnp.float32)
    acc += jnp.dot(t_ref[...], w2_ref[...], preferred_element_type=jnp.float32)
    acc += jnp.dot(v_ref[...], w3_ref[...], preferred_element_type=jnp.float32)
    o_ref[...] = acc + b_ref[...]


def _tc_fuse(g, t, v, w1, w2, w3, b2):
    bm = 2048
    grid = (BATCH // bm,)
    return pl.pallas_call(
        _fuse_body,
        grid=grid,
        in_specs=[
            pl.BlockSpec((bm, EMBED), lambda i: (i, 0)),
            pl.BlockSpec((bm, EMBED), lambda i: (i, 0)),
            pl.BlockSpec((bm, EMBED), lambda i: (i, 0)),
            pl.BlockSpec((EMBED, LATENT), lambda i: (0, 0)),
            pl.BlockSpec((EMBED, LATENT), lambda i: (0, 0)),
            pl.BlockSpec((EMBED, LATENT), lambda i: (0, 0)),
            pl.BlockSpec((1, LATENT), lambda i: (0, 0)),
        ],
        out_specs=pl.BlockSpec((bm, LATENT), lambda i: (i, 0)),
        out_shape=jax.ShapeDtypeStruct((BATCH, LATENT), jnp.float32),
    )(g, t, v, w1, w2, w3, b2)


@jax.jit
def kernel(group_indices, txt_embed, vision_embed, table, W, b):
    idx3 = group_indices.astype(jnp.int32).reshape(_NW, _NCH, _CHUNK)
    g = _sc_gather(table, idx3)
    w1 = W[:EMBED]
    w2 = W[EMBED:2 * EMBED]
    w3 = W[2 * EMBED:]
    return _tc_fuse(g, txt_embed, vision_embed, w1, w2, w3, b.reshape(1, LATENT))


# speedup vs baseline: 1.6932x; 1.6932x over previous
import functools

import jax
import jax.numpy as jnp
from jax import lax
from jax.experimental import pallas as pl
from jax.experimental.pallas import tpu as pltpu
from jax.experimental.pallas import tpu_sc as plsc

GROUP_NUM = 1000000
EMBED = 64
LATENT = 128
BATCH = 16384

_NC = 2
_NS = 16
_NW = _NC * _NS
_B_PER_W = BATCH // _NW
_CHUNK = 128
_NCH = _B_PER_W // _CHUNK


_UNROLL = 16


def _sc_gather(table, idx2):
    mesh = plsc.VectorSubcoreMesh(core_axis_name="c", subcore_axis_name="s")

    @functools.partial(
        pl.kernel,
        mesh=mesh,
        out_type=jax.ShapeDtypeStruct((BATCH, EMBED), jnp.float32),
        scratch_types=[
            pltpu.VMEM((_B_PER_W,), jnp.int32),
            pltpu.VMEM((_B_PER_W, EMBED), jnp.float32),
            pltpu.SemaphoreType.DMA,
        ],
    )
    def k(table_hbm, idx_hbm, out_hbm, idx_v, rows_v, sem):
        wid = lax.axis_index("s") * _NC + lax.axis_index("c")
        base = wid * _B_PER_W
        pltpu.sync_copy(idx_hbm.at[wid], idx_v)

        def body(j, carry):
            vec = idx_v[pl.ds(j * _UNROLL, _UNROLL)]
            for u in range(_UNROLL):
                r = vec[u]
                pltpu.async_copy(
                    table_hbm.at[pl.ds(r, 1)],
                    rows_v.at[pl.ds(j * _UNROLL + u, 1)],
                    sem,
                )
            return carry

        lax.fori_loop(0, _B_PER_W // _UNROLL, body, 0)
        pltpu.make_async_copy(
            table_hbm.at[pl.ds(0, _B_PER_W)], rows_v, sem
        ).wait()
        pltpu.sync_copy(rows_v, out_hbm.at[pl.ds(base, _B_PER_W)])

    return k(table, idx2)


def _fuse_body(g_ref, t_ref, v_ref, w1_ref, w2_ref, w3_ref, b_ref, o_ref):
    acc = jnp.dot(g_ref[...], w1_ref[...], preferred_element_type=jnp.float32)
    acc += jnp.dot(t_ref[...], w2_ref[...], preferred_element_type=jnp.float32)
    acc += jnp.dot(v_ref[...], w3_ref[...], preferred_element_type=jnp.float32)
    o_ref[...] = acc + b_ref[...]


def _tc_fuse(g, t, v, w1, w2, w3, b2):
    bm = 2048
    grid = (BATCH // bm,)
    return pl.pallas_call(
        _fuse_body,
        grid=grid,
        in_specs=[
            pl.BlockSpec((bm, EMBED), lambda i: (i, 0)),
            pl.BlockSpec((bm, EMBED), lambda i: (i, 0)),
            pl.BlockSpec((bm, EMBED), lambda i: (i, 0)),
            pl.BlockSpec((EMBED, LATENT), lambda i: (0, 0)),
            pl.BlockSpec((EMBED, LATENT), lambda i: (0, 0)),
            pl.BlockSpec((EMBED, LATENT), lambda i: (0, 0)),
            pl.BlockSpec((1, LATENT), lambda i: (0, 0)),
        ],
        out_specs=pl.BlockSpec((bm, LATENT), lambda i: (i, 0)),
        out_shape=jax.ShapeDtypeStruct((BATCH, LATENT), jnp.float32),
    )(g, t, v, w1, w2, w3, b2)


@jax.jit
def kernel(group_indices, txt_embed, vision_embed, table, W, b):
    idx2 = group_indices.astype(jnp.int32).reshape(_NW, _B_PER_W)
    g = _sc_gather(table, idx2)
    w1 = W[:EMBED]
    w2 = W[EMBED:2 * EMBED]
    w3 = W[2 * EMBED:]
    return _tc_fuse(g, txt_embed, vision_embed, w1, w2, w3, b.reshape(1, LATENT))
